# baseline (speedup 1.0000x reference)
import functools

import jax
import jax.numpy as jnp
from jax import lax
from jax.experimental import pallas as pl
from jax.experimental.pallas import tpu as pltpu
from jax.experimental.pallas import tpu_sc as plsc

N_NODES = 10000
N_EDGES = 320000
D = 128

NC = 2
NS = 16
NW = NC * NS
L = 16

EB = 128
BPW = ((N_EDGES + EB * NW - 1) // (EB * NW) + 7) // 8 * 8
NBLK = BPW * NW
EPAD = NBLK * EB
NP = 10112
RPT = NP // NS


def _fill_rows(rows_v, value):
    vv = jnp.full((L,), value, jnp.float32)

    def frow(r, carry):
        for j in range(D // L):
            rows_v[r, pl.ds(j * L, L)] = vv
        return carry

    lax.fori_loop(0, EB, frow, 0)


def _zero_acc_slice(rows_v, acc_sh, row0):
    for k in range(RPT // EB):
        pltpu.sync_copy(rows_v, acc_sh.at[pl.ds(row0 + k * EB, EB)])
    rem = RPT % EB
    if rem:
        pltpu.sync_copy(rows_v.at[pl.ds(0, rem)],
                        acc_sh.at[pl.ds(row0 + (RPT // EB) * EB, rem)])


@functools.lru_cache(maxsize=None)
def _make_sc_agg(with_counts):
    mesh = plsc.VectorSubcoreMesh(core_axis_name="c", subcore_axis_name="s")

    def body(x_hbm, src_hbm, dst_hbm, *rest):
        if with_counts:
            out_hbm, cnt_hbm, src_v, dst_v, rows_v, acc_sh, sem = rest
        else:
            out_hbm, src_v, dst_v, rows_v, acc_sh, sem = rest
        cid = lax.axis_index("c")
        sid = lax.axis_index("s")
        wid = cid * NS + sid
        row0 = sid * RPT

        _fill_rows(rows_v, 0.0)
        _zero_acc_slice(rows_v, acc_sh, row0)

        pltpu.sync_copy(src_hbm.at[pl.ds(wid * BPW, BPW)], src_v)
        pltpu.sync_copy(dst_hbm.at[pl.ds(wid * BPW, BPW)], dst_v)
        plsc.subcore_barrier()

        def blk(i, carry):
            pltpu.async_copy(x_hbm.at[src_v.at[i]], rows_v, sem).wait()
            pltpu.sync_copy(rows_v, acc_sh.at[dst_v.at[i]], add=True)
            return carry

        lax.fori_loop(0, BPW, blk, 0)
        plsc.subcore_barrier()

        pltpu.sync_copy(acc_sh.at[pl.ds(row0, RPT)],
                        out_hbm.at[cid, pl.ds(row0, RPT)])

        if with_counts:
            plsc.subcore_barrier()
            _fill_rows(rows_v, 0.0)
            _zero_acc_slice(rows_v, acc_sh, row0)
            _fill_rows(rows_v, 1.0)
            plsc.subcore_barrier()

            def cblk(i, carry):
                pltpu.sync_copy(rows_v, acc_sh.at[dst_v.at[i]], add=True)
                return carry

            lax.fori_loop(0, BPW, cblk, 0)
            plsc.subcore_barrier()
            pltpu.sync_copy(acc_sh.at[pl.ds(row0, RPT)],
                            cnt_hbm.at[cid, pl.ds(row0, RPT)])

    shape = jax.ShapeDtypeStruct((NC, NP, D), jnp.float32)
    return pl.kernel(
        body,
        out_type=(shape, shape) if with_counts else shape,
        mesh=mesh,
        scratch_types=[
            pltpu.VMEM((BPW, EB), jnp.int32),
            pltpu.VMEM((BPW, EB), jnp.int32),
            pltpu.VMEM((EB, D), jnp.float32),
            pltpu.VMEM_SHARED((NP, D), jnp.float32),
            pltpu.SemaphoreType.DMA,
        ],
    )


def _dense1_body(p_ref, c_ref, x_ref, wl_ref, bl_ref, wr_ref, g_ref, be_ref,
                 h_ref, ic_ref):
    s = p_ref[0, :N_NODES, :] + p_ref[1, :N_NODES, :]
    cnt = c_ref[0, :N_NODES, 0:1] + c_ref[1, :N_NODES, 0:1]
    inv = 1.0 / jnp.maximum(cnt, 1.0)
    mean = s * inv
    out = (lax.dot_general(mean, wl_ref[...], (((1,), (1,)), ((), ())),
                           preferred_element_type=jnp.float32)
           + bl_ref[...]
           + lax.dot_general(x_ref[...], wr_ref[...], (((1,), (1,)), ((), ())),
                             preferred_element_type=jnp.float32))
    nrm = jnp.sqrt(jnp.sum(out * out, axis=1, keepdims=True))
    out = out / jnp.maximum(nrm, 1e-12)
    mu = jnp.mean(out, axis=0, keepdims=True)
    var = jnp.mean((out - mu) ** 2, axis=0, keepdims=True)
    out = (out - mu) * lax.rsqrt(var + 1e-5) * g_ref[...] + be_ref[...]
    h_ref[...] = jnp.maximum(out, 0.0)
    ic_ref[...] = inv


def _dense2_body(p_ref, h_ref, ic_ref, wl_ref, bl_ref, wr_ref, o_ref):
    s = p_ref[0, :N_NODES, :] + p_ref[1, :N_NODES, :]
    mean = s * ic_ref[...]
    out = (lax.dot_general(mean, wl_ref[...], (((1,), (1,)), ((), ())),
                           preferred_element_type=jnp.float32)
           + bl_ref[...]
           + lax.dot_general(h_ref[...], wr_ref[...], (((1,), (1,)), ((), ())),
                             preferred_element_type=jnp.float32))
    nrm = jnp.sqrt(jnp.sum(out * out, axis=1, keepdims=True))
    o_ref[...] = out / jnp.maximum(nrm, 1e-12)


_dense1 = pl.pallas_call(
    _dense1_body,
    out_shape=(jax.ShapeDtypeStruct((N_NODES, D), jnp.float32),
               jax.ShapeDtypeStruct((N_NODES, 1), jnp.float32)),
)

_dense2 = pl.pallas_call(
    _dense2_body,
    out_shape=jax.ShapeDtypeStruct((N_NODES, D), jnp.float32),
)


def _copy_body(i_ref, o_ref):
    o_ref[...] = i_ref[...]


_tc_copy = pl.pallas_call(
    _copy_body,
    out_shape=jax.ShapeDtypeStruct((N_NODES, D), jnp.float32),
)


def kernel(x, edge_index, W1l, b1l, W1r, W2l, b2l, W2r, gamma, beta):
    src = edge_index[0].astype(jnp.int32)
    dst = edge_index[1].astype(jnp.int32)
    pad = EPAD - N_EDGES
    srcp = jnp.concatenate([src, jnp.zeros((pad,), jnp.int32)]).reshape(
        NBLK, EB)
    dstp = jnp.concatenate([dst, jnp.full((pad,), N_NODES, jnp.int32)]
                           ).reshape(NBLK, EB)

    xc = _tc_copy(x)
    p1, cnts = _make_sc_agg(True)(xc, srcp, dstp)
    h, inv = _dense1(p1, cnts, x, W1l, b1l.reshape(1, D), W1r,
                     gamma.reshape(1, D), beta.reshape(1, D))
    p2 = _make_sc_agg(False)(h, srcp, dstp)
    out = _dense2(p2, h, inv, W2l, b2l.reshape(1, D), W2r)
    return out


# speedup vs baseline: 1.0736x; 1.0736x over previous
import functools

import jax
import jax.numpy as jnp
from jax import lax
from jax.experimental import pallas as pl
from jax.experimental.pallas import tpu as pltpu
from jax.experimental.pallas import tpu_sc as plsc

N_NODES = 10000
N_EDGES = 320000
D = 128

NC = 2
NS = 16
NW = NC * NS
L = 16

EB = 128
NBLK = 2560
EPAD = NBLK * EB
NP = 10112
RPT = NP // NS

FAST_CID = 1
BF = 112
BS = 48
NBLK_ALLOC = NBLK + BF - BS


def _fill_rows(rows_v, value):
    vv = jnp.full((L,), value, jnp.float32)

    def frow(r, carry):
        for j in range(D // L):
            rows_v[r, pl.ds(j * L, L)] = vv
        return carry

    lax.fori_loop(0, EB, frow, 0)


def _zero_acc_slice(rows_v, acc_sh, row0):
    for k in range(RPT // EB):
        pltpu.sync_copy(rows_v, acc_sh.at[pl.ds(row0 + k * EB, EB)])
    rem = RPT % EB
    if rem:
        pltpu.sync_copy(rows_v.at[pl.ds(0, rem)],
                        acc_sh.at[pl.ds(row0 + (RPT // EB) * EB, rem)])


@functools.lru_cache(maxsize=None)
def _make_sc_agg(with_counts):
    mesh = plsc.VectorSubcoreMesh(core_axis_name="c", subcore_axis_name="s")

    def body(x_hbm, src_hbm, dst_hbm, *rest):
        if with_counts:
            out_hbm, cnt_hbm, src_v, dst_v, rows_v, acc_sh, sem = rest
        else:
            out_hbm, src_v, dst_v, rows_v, acc_sh, sem = rest
        cid = lax.axis_index("c")
        sid = lax.axis_index("s")
        row0 = sid * RPT
        nblk = jnp.where(cid == FAST_CID, BF, BS)
        base = pl.multiple_of(
            jnp.where(cid == FAST_CID, sid * BF, NS * BF + sid * BS), 8)

        _fill_rows(rows_v, 0.0)
        _zero_acc_slice(rows_v, acc_sh, row0)

        pltpu.sync_copy(src_hbm.at[pl.ds(base, BF)], src_v)
        pltpu.sync_copy(dst_hbm.at[pl.ds(base, BF)], dst_v)
        plsc.subcore_barrier()

        def blk(i, carry):
            pltpu.async_copy(x_hbm.at[src_v.at[i]], rows_v, sem).wait()
            pltpu.sync_copy(rows_v, acc_sh.at[dst_v.at[i]], add=True)
            return carry

        lax.fori_loop(0, nblk, blk, 0)
        plsc.subcore_barrier()

        pltpu.sync_copy(acc_sh.at[pl.ds(row0, RPT)],
                        out_hbm.at[cid, pl.ds(row0, RPT)])

        if with_counts:
            plsc.subcore_barrier()
            _fill_rows(rows_v, 0.0)
            _zero_acc_slice(rows_v, acc_sh, row0)
            _fill_rows(rows_v, 1.0)
            plsc.subcore_barrier()

            def cblk(i, carry):
                pltpu.sync_copy(rows_v, acc_sh.at[dst_v.at[i]], add=True)
                return carry

            lax.fori_loop(0, nblk, cblk, 0)
            plsc.subcore_barrier()
            pltpu.sync_copy(acc_sh.at[pl.ds(row0, RPT)],
                            cnt_hbm.at[cid, pl.ds(row0, RPT)])

    shape = jax.ShapeDtypeStruct((NC, NP, D), jnp.float32)
    return pl.kernel(
        body,
        out_type=(shape, shape) if with_counts else shape,
        mesh=mesh,
        scratch_types=[
            pltpu.VMEM((BF, EB), jnp.int32),
            pltpu.VMEM((BF, EB), jnp.int32),
            pltpu.VMEM((EB, D), jnp.float32),
            pltpu.VMEM_SHARED((NP, D), jnp.float32),
            pltpu.SemaphoreType.DMA,
        ],
    )


def _dense1_body(p_ref, c_ref, x_ref, wl_ref, bl_ref, wr_ref, g_ref, be_ref,
                 h_ref, ic_ref):
    s = p_ref[0, :N_NODES, :] + p_ref[1, :N_NODES, :]
    cnt = c_ref[0, :N_NODES, 0:1] + c_ref[1, :N_NODES, 0:1]
    inv = 1.0 / jnp.maximum(cnt, 1.0)
    mean = s * inv
    out = (lax.dot_general(mean, wl_ref[...], (((1,), (1,)), ((), ())),
                           preferred_element_type=jnp.float32)
           + bl_ref[...]
           + lax.dot_general(x_ref[...], wr_ref[...], (((1,), (1,)), ((), ())),
                             preferred_element_type=jnp.float32))
    nrm = jnp.sqrt(jnp.sum(out * out, axis=1, keepdims=True))
    out = out / jnp.maximum(nrm, 1e-12)
    mu = jnp.mean(out, axis=0, keepdims=True)
    var = jnp.mean((out - mu) ** 2, axis=0, keepdims=True)
    out = (out - mu) * lax.rsqrt(var + 1e-5) * g_ref[...] + be_ref[...]
    h_ref[...] = jnp.maximum(out, 0.0)
    ic_ref[...] = inv


def _dense2_body(p_ref, h_ref, ic_ref, wl_ref, bl_ref, wr_ref, o_ref):
    s = p_ref[0, :N_NODES, :] + p_ref[1, :N_NODES, :]
    mean = s * ic_ref[...]
    out = (lax.dot_general(mean, wl_ref[...], (((1,), (1,)), ((), ())),
                           preferred_element_type=jnp.float32)
           + bl_ref[...]
           + lax.dot_general(h_ref[...], wr_ref[...], (((1,), (1,)), ((), ())),
                             preferred_element_type=jnp.float32))
    nrm = jnp.sqrt(jnp.sum(out * out, axis=1, keepdims=True))
    o_ref[...] = out / jnp.maximum(nrm, 1e-12)


_dense1 = pl.pallas_call(
    _dense1_body,
    out_shape=(jax.ShapeDtypeStruct((N_NODES, D), jnp.float32),
               jax.ShapeDtypeStruct((N_NODES, 1), jnp.float32)),
)

_dense2 = pl.pallas_call(
    _dense2_body,
    out_shape=jax.ShapeDtypeStruct((N_NODES, D), jnp.float32),
)


def kernel(x, edge_index, W1l, b1l, W1r, W2l, b2l, W2r, gamma, beta):
    src = edge_index[0].astype(jnp.int32)
    dst = edge_index[1].astype(jnp.int32)
    pad = NBLK_ALLOC * EB - N_EDGES
    srcp = jnp.concatenate([src, jnp.zeros((pad,), jnp.int32)]).reshape(
        NBLK_ALLOC, EB)
    dstp = jnp.concatenate([dst, jnp.full((pad,), N_NODES, jnp.int32)]
                           ).reshape(NBLK_ALLOC, EB)

    p1, cnts = _make_sc_agg(True)(x, srcp, dstp)
    h, inv = _dense1(p1, cnts, x, W1l, b1l.reshape(1, D), W1r,
                     gamma.reshape(1, D), beta.reshape(1, D))
    p2 = _make_sc_agg(False)(h, srcp, dstp)
    out = _dense2(p2, h, inv, W2l, b2l.reshape(1, D), W2r)
    return out


# speedup vs baseline: 1.0813x; 1.0072x over previous
import functools

import jax
import jax.numpy as jnp
from jax import lax
from jax.experimental import pallas as pl
from jax.experimental.pallas import tpu as pltpu
from jax.experimental.pallas import tpu_sc as plsc

N_NODES = 10000
N_EDGES = 320000
D = 128

NC = 2
NS = 16
NW = NC * NS
L = 16

EB = 128
NBLK = 2560
EPAD = NBLK * EB
NP = 10112
RPT = NP // NS

FAST_CID = 0
BF = 112
BS = 48
NBLK_ALLOC = NBLK + BF - BS


def _fill_rows(rows_v, value):
    vv = jnp.full((L,), value, jnp.float32)

    def frow(r, carry):
        for j in range(D // L):
            rows_v[r, pl.ds(j * L, L)] = vv
        return carry

    lax.fori_loop(0, EB, frow, 0)


def _zero_acc_slice(rows_v, acc_sh, row0):
    for k in range(RPT // EB):
        pltpu.sync_copy(rows_v, acc_sh.at[pl.ds(row0 + k * EB, EB)])
    rem = RPT % EB
    if rem:
        pltpu.sync_copy(rows_v.at[pl.ds(0, rem)],
                        acc_sh.at[pl.ds(row0 + (RPT // EB) * EB, rem)])


@functools.lru_cache(maxsize=None)
def _make_sc_agg(with_counts):
    mesh = plsc.VectorSubcoreMesh(core_axis_name="c", subcore_axis_name="s")

    def body(x_hbm, src_hbm, dst_hbm, *rest):
        if with_counts:
            out_hbm, cnt_hbm, src_v, dst_v, rows_v, acc_sh, sem = rest
        else:
            out_hbm, src_v, dst_v, rows_v, acc_sh, sem = rest
        cid = lax.axis_index("c")
        sid = lax.axis_index("s")
        row0 = sid * RPT
        nblk = jnp.where(cid == FAST_CID, BF, BS)
        base = pl.multiple_of(
            jnp.where(cid == FAST_CID, sid * BF, NS * BF + sid * BS), 8)

        _fill_rows(rows_v, 0.0)
        _zero_acc_slice(rows_v, acc_sh, row0)

        pltpu.sync_copy(src_hbm.at[pl.ds(base, BF)], src_v)
        pltpu.sync_copy(dst_hbm.at[pl.ds(base, BF)], dst_v)
        plsc.subcore_barrier()

        def blk(i, carry):
            pltpu.async_copy(x_hbm.at[src_v.at[i]], rows_v, sem).wait()
            pltpu.sync_copy(rows_v, acc_sh.at[dst_v.at[i]], add=True)
            return carry

        lax.fori_loop(0, nblk, blk, 0)
        plsc.subcore_barrier()

        pltpu.sync_copy(acc_sh.at[pl.ds(row0, RPT)],
                        out_hbm.at[cid, pl.ds(row0, RPT)])

        if with_counts:
            plsc.subcore_barrier()
            _fill_rows(rows_v, 0.0)
            _zero_acc_slice(rows_v, acc_sh, row0)
            _fill_rows(rows_v, 1.0)
            plsc.subcore_barrier()

            def cblk(i, carry):
                pltpu.sync_copy(rows_v, acc_sh.at[dst_v.at[i]], add=True)
                return carry

            lax.fori_loop(0, nblk, cblk, 0)
            plsc.subcore_barrier()
            pltpu.sync_copy(acc_sh.at[pl.ds(row0, RPT)],
                            cnt_hbm.at[cid, pl.ds(row0, RPT)])

    shape = jax.ShapeDtypeStruct((NC, NP, D), jnp.float32)
    return pl.kernel(
        body,
        out_type=(shape, shape) if with_counts else shape,
        mesh=mesh,
        scratch_types=[
            pltpu.VMEM((BF, EB), jnp.int32),
            pltpu.VMEM((BF, EB), jnp.int32),
            pltpu.VMEM((EB, D), jnp.float32),
            pltpu.VMEM_SHARED((NP, D), jnp.float32),
            pltpu.SemaphoreType.DMA,
        ],
    )


def _dense1_body(p_ref, c_ref, x_ref, wl_ref, bl_ref, wr_ref, g_ref, be_ref,
                 h_ref, ic_ref):
    s = p_ref[0, :N_NODES, :] + p_ref[1, :N_NODES, :]
    cnt = c_ref[0, :N_NODES, 0:1] + c_ref[1, :N_NODES, 0:1]
    inv = 1.0 / jnp.maximum(cnt, 1.0)
    mean = s * inv
    out = (lax.dot_general(mean, wl_ref[...], (((1,), (1,)), ((), ())),
                           preferred_element_type=jnp.float32)
           + bl_ref[...]
           + lax.dot_general(x_ref[...], wr_ref[...], (((1,), (1,)), ((), ())),
                             preferred_element_type=jnp.float32))
    nrm = jnp.sqrt(jnp.sum(out * out, axis=1, keepdims=True))
    out = out / jnp.maximum(nrm, 1e-12)
    mu = jnp.mean(out, axis=0, keepdims=True)
    var = jnp.mean((out - mu) ** 2, axis=0, keepdims=True)
    out = (out - mu) * lax.rsqrt(var + 1e-5) * g_ref[...] + be_ref[...]
    h_ref[...] = jnp.maximum(out, 0.0)
    ic_ref[...] = inv


def _dense2_body(p_ref, h_ref, ic_ref, wl_ref, bl_ref, wr_ref, o_ref):
    s = p_ref[0, :N_NODES, :] + p_ref[1, :N_NODES, :]
    mean = s * ic_ref[...]
    out = (lax.dot_general(mean, wl_ref[...], (((1,), (1,)), ((), ())),
                           preferred_element_type=jnp.float32)
           + bl_ref[...]
           + lax.dot_general(h_ref[...], wr_ref[...], (((1,), (1,)), ((), ())),
                             preferred_element_type=jnp.float32))
    nrm = jnp.sqrt(jnp.sum(out * out, axis=1, keepdims=True))
    o_ref[...] = out / jnp.maximum(nrm, 1e-12)


_dense1 = pl.pallas_call(
    _dense1_body,
    out_shape=(jax.ShapeDtypeStruct((N_NODES, D), jnp.float32),
               jax.ShapeDtypeStruct((N_NODES, 1), jnp.float32)),
)

_dense2 = pl.pallas_call(
    _dense2_body,
    out_shape=jax.ShapeDtypeStruct((N_NODES, D), jnp.float32),
)


def kernel(x, edge_index, W1l, b1l, W1r, W2l, b2l, W2r, gamma, beta):
    src = edge_index[0].astype(jnp.int32)
    dst = edge_index[1].astype(jnp.int32)
    pad = NBLK_ALLOC * EB - N_EDGES
    srcp = jnp.concatenate([src, jnp.zeros((pad,), jnp.int32)]).reshape(
        NBLK_ALLOC, EB)
    dstp = jnp.concatenate([dst, jnp.full((pad,), N_NODES, jnp.int32)]
                           ).reshape(NBLK_ALLOC, EB)

    p1, cnts = _make_sc_agg(True)(x, srcp, dstp)
    h, inv = _dense1(p1, cnts, x, W1l, b1l.reshape(1, D), W1r,
                     gamma.reshape(1, D), beta.reshape(1, D))
    p2 = _make_sc_agg(False)(h, srcp, dstp)
    out = _dense2(p2, h, inv, W2l, b2l.reshape(1, D), W2r)
    return out


# speedup vs baseline: 1.0825x; 1.0011x over previous
import functools

import jax
import jax.numpy as jnp
from jax import lax
from jax.experimental import pallas as pl
from jax.experimental.pallas import tpu as pltpu
from jax.experimental.pallas import tpu_sc as plsc

N_NODES = 10000
N_EDGES = 320000
D = 128

NC = 2
NS = 16
NW = NC * NS
L = 16

EB = 128
NBLK = 2560
EPAD = NBLK * EB
NP = 10112
RPT = NP // NS

FAST_CID = 0
BF = 112
BS = 48
NBLK_ALLOC = NBLK + BF - BS


def _fill_rows(rows_v, value):
    vv = jnp.full((L,), value, jnp.float32)

    def frow(r, carry):
        for j in range(D // L):
            rows_v[r, pl.ds(j * L, L)] = vv
        return carry

    lax.fori_loop(0, EB, frow, 0)


def _zero_acc_slice(rows_v, acc_sh, row0):
    for k in range(RPT // EB):
        pltpu.sync_copy(rows_v, acc_sh.at[pl.ds(row0 + k * EB, EB)])
    rem = RPT % EB
    if rem:
        pltpu.sync_copy(rows_v.at[pl.ds(0, rem)],
                        acc_sh.at[pl.ds(row0 + (RPT // EB) * EB, rem)])


@functools.lru_cache(maxsize=None)
def _make_sc_agg(with_counts):
    mesh = plsc.VectorSubcoreMesh(core_axis_name="c", subcore_axis_name="s")

    def body(x_hbm, src_hbm, dst_hbm, *rest):
        if with_counts:
            out_hbm, cnt_hbm, src_v, dst_v, rows_v, acc_sh, sem = rest
        else:
            out_hbm, src_v, dst_v, rows_v, acc_sh, sem = rest
        cid = lax.axis_index("c")
        sid = lax.axis_index("s")
        row0 = sid * RPT
        nblk = jnp.where(cid == FAST_CID, BF, BS)
        base = pl.multiple_of(
            jnp.where(cid == FAST_CID, sid * BF, NS * BF + sid * BS), 8)

        _fill_rows(rows_v, 0.0)
        _zero_acc_slice(rows_v, acc_sh, row0)

        pltpu.sync_copy(src_hbm.at[pl.ds(base, BF)], src_v)
        pltpu.sync_copy(dst_hbm.at[pl.ds(base, BF)], dst_v)
        plsc.subcore_barrier()

        def blk(i, carry):
            pltpu.async_copy(x_hbm.at[src_v.at[i]], rows_v, sem).wait()
            pltpu.sync_copy(rows_v, acc_sh.at[dst_v.at[i]], add=True)
            return carry

        lax.fori_loop(0, nblk, blk, 0)
        plsc.subcore_barrier()

        pltpu.sync_copy(acc_sh.at[pl.ds(row0, RPT)],
                        out_hbm.at[cid, pl.ds(row0, RPT)])

        if with_counts:
            plsc.subcore_barrier()
            _fill_rows(rows_v, 0.0)
            _zero_acc_slice(rows_v, acc_sh, row0)
            _fill_rows(rows_v, 1.0)
            plsc.subcore_barrier()

            def cblk(i, carry):
                pltpu.async_copy(rows_v, acc_sh.at[dst_v.at[i]], sem,
                                 add=True)
                return carry

            lax.fori_loop(0, nblk, cblk, 0)

            def cdrain(i, carry):
                pltpu.make_async_copy(rows_v, acc_sh.at[dst_v.at[0]],
                                      sem).wait()
                return carry

            lax.fori_loop(0, nblk, cdrain, 0)
            plsc.subcore_barrier()
            pltpu.sync_copy(acc_sh.at[pl.ds(row0, RPT)],
                            cnt_hbm.at[cid, pl.ds(row0, RPT)])

    shape = jax.ShapeDtypeStruct((NC, NP, D), jnp.float32)
    return pl.kernel(
        body,
        out_type=(shape, shape) if with_counts else shape,
        mesh=mesh,
        scratch_types=[
            pltpu.VMEM((BF, EB), jnp.int32),
            pltpu.VMEM((BF, EB), jnp.int32),
            pltpu.VMEM((EB, D), jnp.float32),
            pltpu.VMEM_SHARED((NP, D), jnp.float32),
            pltpu.SemaphoreType.DMA,
        ],
    )


def _dense1_body(p_ref, c_ref, x_ref, wl_ref, bl_ref, wr_ref, g_ref, be_ref,
                 h_ref, ic_ref):
    s = p_ref[0, :N_NODES, :] + p_ref[1, :N_NODES, :]
    cnt = c_ref[0, :N_NODES, 0:1] + c_ref[1, :N_NODES, 0:1]
    inv = 1.0 / jnp.maximum(cnt, 1.0)
    mean = s * inv
    out = (lax.dot_general(mean, wl_ref[...], (((1,), (1,)), ((), ())),
                           preferred_element_type=jnp.float32)
           + bl_ref[...]
           + lax.dot_general(x_ref[...], wr_ref[...], (((1,), (1,)), ((), ())),
                             preferred_element_type=jnp.float32))
    nrm = jnp.sqrt(jnp.sum(out * out, axis=1, keepdims=True))
    out = out / jnp.maximum(nrm, 1e-12)
    mu = jnp.mean(out, axis=0, keepdims=True)
    var = jnp.mean((out - mu) ** 2, axis=0, keepdims=True)
    out = (out - mu) * lax.rsqrt(var + 1e-5) * g_ref[...] + be_ref[...]
    h_ref[...] = jnp.maximum(out, 0.0)
    ic_ref[...] = inv


def _dense2_body(p_ref, h_ref, ic_ref, wl_ref, bl_ref, wr_ref, o_ref):
    s = p_ref[0, :N_NODES, :] + p_ref[1, :N_NODES, :]
    mean = s * ic_ref[...]
    out = (lax.dot_general(mean, wl_ref[...], (((1,), (1,)), ((), ())),
                           preferred_element_type=jnp.float32)
           + bl_ref[...]
           + lax.dot_general(h_ref[...], wr_ref[...], (((1,), (1,)), ((), ())),
                             preferred_element_type=jnp.float32))
    nrm = jnp.sqrt(jnp.sum(out * out, axis=1, keepdims=True))
    o_ref[...] = out / jnp.maximum(nrm, 1e-12)


_dense1 = pl.pallas_call(
    _dense1_body,
    out_shape=(jax.ShapeDtypeStruct((N_NODES, D), jnp.float32),
               jax.ShapeDtypeStruct((N_NODES, 1), jnp.float32)),
)

_dense2 = pl.pallas_call(
    _dense2_body,
    out_shape=jax.ShapeDtypeStruct((N_NODES, D), jnp.float32),
)


def kernel(x, edge_index, W1l, b1l, W1r, W2l, b2l, W2r, gamma, beta):
    src = edge_index[0].astype(jnp.int32)
    dst = edge_index[1].astype(jnp.int32)
    pad = NBLK_ALLOC * EB - N_EDGES
    srcp = jnp.concatenate([src, jnp.zeros((pad,), jnp.int32)]).reshape(
        NBLK_ALLOC, EB)
    dstp = jnp.concatenate([dst, jnp.full((pad,), N_NODES, jnp.int32)]
                           ).reshape(NBLK_ALLOC, EB)

    p1, cnts = _make_sc_agg(True)(x, srcp, dstp)
    h, inv = _dense1(p1, cnts, x, W1l, b1l.reshape(1, D), W1r,
                     gamma.reshape(1, D), beta.reshape(1, D))
    p2 = _make_sc_agg(False)(h, srcp, dstp)
    out = _dense2(p2, h, inv, W2l, b2l.reshape(1, D), W2r)
    return out
